# baseline (speedup 1.0000x reference)
import functools

import jax
import jax.numpy as jnp
from jax import lax
from jax.experimental import pallas as pl
from jax.experimental.pallas import tpu as pltpu
from jax.experimental.pallas import tpu_sc as plsc

NC = 2
NS = 16
NW = NC * NS
LANES = 16
H = 128
BLK = 256
CK = 128
NB = 2
LA = 1
PSHIFT = 14

_HIGH = lax.Precision.HIGHEST


def _largest_div(n, cap):
    for z in range(min(n, cap), 0, -1):
        if n % z == 0:
            return z
    return 1


DDEG = 16


def _make_deg_kernel(CHD, NPAD, NACC):
    rpt = NACC // NS
    zr = _largest_div(rpt, 96)
    opt = NPAD // NS
    mesh = plsc.VectorSubcoreMesh(core_axis_name="c", subcore_axis_name="s")

    @functools.partial(
        pl.kernel,
        mesh=mesh,
        out_type=jax.ShapeDtypeStruct((NC, NPAD, DDEG), jnp.float32),
        scratch_types=[
            pltpu.VMEM((CHD, 128), jnp.int32),
            pltpu.VMEM((128, DDEG), jnp.float32),
            pltpu.VMEM((zr, DDEG), jnp.float32),
            pltpu.VMEM_SHARED((NACC, DDEG), jnp.float32),
        ],
    )
    def deg_kernel(col_hbm, out_hbm, col_scr, ones_v, zeros_v, acc):
        c = lax.axis_index("c")
        s = lax.axis_index("s")
        w = c * NS + s

        def fill_ones(i, _):
            ones_v[i] = jnp.ones((LANES,), jnp.float32)
            return 0

        lax.fori_loop(0, 128, fill_ones, 0)

        def fill_zeros(i, _):
            zeros_v[i] = jnp.zeros((LANES,), jnp.float32)
            return 0

        lax.fori_loop(0, zr, fill_zeros, 0)

        def zero_acc(k, _):
            pltpu.sync_copy(zeros_v, acc.at[pl.ds(s * rpt + k * zr, zr)])
            return 0

        lax.fori_loop(0, rpt // zr, zero_acc, 0)
        plsc.subcore_barrier()

        pltpu.sync_copy(col_hbm.at[w], col_scr)

        def body(j, _):
            pltpu.sync_copy(ones_v, acc.at[col_scr.at[j]], add=True)
            return 0

        lax.fori_loop(0, CHD, body, 0)
        plsc.subcore_barrier()
        pltpu.sync_copy(acc.at[pl.ds(s * opt, opt)],
                        out_hbm.at[c, pl.ds(s * opt, opt)])

    return deg_kernel


def _make_agg_kernel(CH, NPAD, NACC):
    rpt = NACC // NS
    zr = _largest_div(rpt, 24)
    opt = NPAD // NS
    assert CH % NB == 0
    mesh = plsc.VectorSubcoreMesh(core_axis_name="c", subcore_axis_name="s")

    @functools.partial(
        pl.kernel,
        mesh=mesh,
        out_type=jax.ShapeDtypeStruct((NC, NPAD, H), jnp.float32),
        scratch_types=[
            pltpu.VMEM((CH * CK,), jnp.int32),
            [pltpu.VMEM((1, CK), jnp.int32) for _ in range(NB)],
            [pltpu.VMEM((1, CK), jnp.int32) for _ in range(NB)],
            [pltpu.VMEM((CK, H), jnp.float32) for _ in range(NB)],
            pltpu.VMEM((zr, H), jnp.float32),
            pltpu.VMEM_SHARED((NACC, H), jnp.float32),
            pltpu.SemaphoreType.DMA,
            pltpu.SemaphoreType.DMA,
        ],
    )
    def agg_kernel(y_hbm, pk_hbm, out_hbm,
                   pk_scr, ridx, cidx, bufs, zeros_v, acc, gsem, ssem):
        c = lax.axis_index("c")
        s = lax.axis_index("s")
        w = c * NS + s

        def fill_zeros(n, _):
            i = n // (H // LANES)
            k = n % (H // LANES)
            zeros_v[i, pl.ds(k * LANES, LANES)] = jnp.zeros((LANES,),
                                                            jnp.float32)
            return 0

        lax.fori_loop(0, zr * (H // LANES), fill_zeros, 0)

        def zero_acc(k, _):
            pltpu.sync_copy(zeros_v, acc.at[pl.ds(s * rpt + k * zr, zr)])
            return 0

        lax.fori_loop(0, rpt // zr, zero_acc, 0)

        pltpu.sync_copy(pk_hbm.at[w], pk_scr)

        def unpack(j, b):
            base = pl.multiple_of(j * CK, CK)
            for k in range(CK // LANES):
                p = pk_scr[pl.ds(base + k * LANES, LANES)]
                ridx[b][0, pl.ds(k * LANES, LANES)] = p >> PSHIFT
                cidx[b][0, pl.ds(k * LANES, LANES)] = p & ((1 << PSHIFT) - 1)

        def gather(b):
            pltpu.async_copy(y_hbm.at[ridx[b].at[0]], bufs[b], gsem)

        def gather_wait(b):
            pltpu.make_async_copy(y_hbm.at[ridx[b].at[0]], bufs[b],
                                  gsem).wait()

        def scatter(b):
            pltpu.async_copy(bufs[b], acc.at[cidx[b].at[0]], ssem, add=True)

        def scatter_wait(b):
            pltpu.make_async_copy(bufs[b], acc.at[cidx[b].at[0]],
                                  ssem).wait()

        plsc.subcore_barrier()

        for b in range(LA):
            unpack(b, b)
            gather(b)

        def body(g, _):
            j0 = g * NB
            for b in range(NB):
                j = j0 + b
                bg = (b + LA) % NB
                unpack(j + LA, bg)
                gather(bg)
                gather_wait(b)
                pltpu.sync_copy(bufs[b], acc.at[cidx[b].at[0]], add=True)
            return 0

        lax.fori_loop(0, CH // NB - 1, body, 0)
        for b in range(NB):
            j = CH - NB + b
            if j + LA < CH:
                bg = (b + LA) % NB
                unpack(j + LA, bg)
                gather(bg)
            gather_wait(b)
            pltpu.sync_copy(bufs[b], acc.at[cidx[b].at[0]], add=True)
        plsc.subcore_barrier()
        pltpu.sync_copy(acc.at[pl.ds(s * opt, opt)],
                        out_hbm.at[c, pl.ds(s * opt, opt)])

    return agg_kernel


def _prep_body(deg2, nip, xv, wh, wl, y0):
    deg = deg2[0, :, 0:1] + deg2[1, :, 0:1] + 1.0
    dinv = lax.rsqrt(deg)
    u = jnp.dot(xv[...], wh[...], precision=_HIGH,
                preferred_element_type=jnp.float32)
    y0[...] = dinv * (u + nip[...] * wl[...])


def _step_body(agg2, y, deg2, nip, wh, wl, b, w1, b1, w2p, b2p, out, ynext):
    deg = deg2[0, :, 0:1] + deg2[1, :, 0:1] + 1.0
    dinv = lax.rsqrt(deg)
    emb = dinv * (agg2[0] + agg2[1] + y[...]) + b[...]
    h = jnp.maximum(
        jnp.dot(emb, w1[...], precision=_HIGH,
                preferred_element_type=jnp.float32) + b1[...], 0.0)
    out[...] = jnp.dot(h, w2p[...], precision=_HIGH,
                       preferred_element_type=jnp.float32) + b2p[...]
    ynext[...] = dinv * (jnp.dot(emb, wh[...], precision=_HIGH,
                                 preferred_element_type=jnp.float32)
                         + nip[...] * wl[...])


def _row_spec(w):
    return pl.BlockSpec((BLK, w), lambda i: (i, 0))


def _part_spec(w):
    return pl.BlockSpec((NC, BLK, w), lambda i: (0, i, 0))


def _full_spec(h, w):
    return pl.BlockSpec((h, w), lambda i: (0, 0))


def kernel(node_input, edge_index, X_v, W, b, W1, b1, W2, b2):
    N = node_input.shape[0]
    E = edge_index.shape[1]
    NPAD = -(-N // BLK) * BLK
    NACC = NPAD + 128
    assert NPAD % NS == 0 and NACC % NS == 0
    assert NPAD < (1 << PSHIFT)
    CH = -(-(-(-E // (NW * CK))) // NB) * NB
    EPAD = NW * CH * CK
    CHD = -(-E // (NW * 128))
    EPADD = NW * CHD * 128

    row = edge_index[0]
    col = edge_index[1]
    packed = jnp.concatenate(
        [(row << PSHIFT) | col,
         jnp.full((EPAD - E,), NPAD, jnp.int32)]).reshape(NW, CH * CK)
    colp = jnp.concatenate(
        [col, jnp.full((EPADD - E,), NPAD, col.dtype)]).reshape(NW, CHD, 128)

    nip = jnp.pad(node_input, (0, NPAD - N)).reshape(NPAD, 1)
    xv = X_v.reshape(1, H)
    wh = W[:H]
    wl = W[H:H + 1]
    b_ = b.reshape(1, H)
    b1_ = b1.reshape(1, H)
    w2p = jnp.pad(W2, ((0, 0), (0, H - W2.shape[1])))
    b2p = jnp.pad(b2.reshape(1, -1), ((0, 0), (0, H - b2.shape[0])))

    deg_k = _make_deg_kernel(CHD, NPAD, NACC)
    agg_k = _make_agg_kernel(CH, NPAD, NACC)

    degp = deg_k(colp)

    grid = (NPAD // BLK,)
    y = pl.pallas_call(
        _prep_body,
        grid=grid,
        in_specs=[_part_spec(DDEG), _row_spec(1), _full_spec(1, H),
                  _full_spec(H, H), _full_spec(1, H)],
        out_specs=_row_spec(H),
        out_shape=jax.ShapeDtypeStruct((NPAD, H), jnp.float32),
    )(degp, nip, xv, wh, wl)

    step_call = pl.pallas_call(
        _step_body,
        grid=grid,
        in_specs=[_part_spec(H), _row_spec(H), _part_spec(DDEG), _row_spec(1),
                  _full_spec(H, H), _full_spec(1, H), _full_spec(1, H),
                  _full_spec(H, H), _full_spec(1, H), _full_spec(H, H),
                  _full_spec(1, H)],
        out_specs=[_row_spec(H), _row_spec(H)],
        out_shape=[jax.ShapeDtypeStruct((NPAD, H), jnp.float32),
                   jax.ShapeDtypeStruct((NPAD, H), jnp.float32)],
    )

    outs = []
    for _ in range(3):
        agg = agg_k(y, packed)
        out_full, y = step_call(agg, y, degp, nip, wh, wl, b_, W1, b1_,
                                w2p, b2p)
        outs.append(out_full[:N, 0])
    return jnp.stack(outs, axis=0)


# speedup vs baseline: 1.3811x; 1.3811x over previous
import functools

import jax
import jax.numpy as jnp
from jax import lax
from jax.experimental import pallas as pl
from jax.experimental.pallas import tpu as pltpu
from jax.experimental.pallas import tpu_sc as plsc

NC = 2
NS = 16
NW = NC * NS
LANES = 16
H = 128
BLK = 256
CK = 128
NB = 1
LA = 0
PSHIFT = 14

_HIGH = lax.Precision.HIGHEST


def _largest_div(n, cap):
    for z in range(min(n, cap), 0, -1):
        if n % z == 0:
            return z
    return 1


DDEG = 16


def _make_deg_kernel(CHD, NPAD, NACC):
    rpt = NACC // NS
    zr = _largest_div(rpt, 96)
    opt = NPAD // NS
    mesh = plsc.VectorSubcoreMesh(core_axis_name="c", subcore_axis_name="s")

    @functools.partial(
        pl.kernel,
        mesh=mesh,
        out_type=jax.ShapeDtypeStruct((NC, NPAD, DDEG), jnp.float32),
        scratch_types=[
            pltpu.VMEM((CHD, 128), jnp.int32),
            pltpu.VMEM((128, DDEG), jnp.float32),
            pltpu.VMEM((zr, DDEG), jnp.float32),
            pltpu.VMEM_SHARED((NACC, DDEG), jnp.float32),
        ],
    )
    def deg_kernel(col_hbm, out_hbm, col_scr, ones_v, zeros_v, acc):
        c = lax.axis_index("c")
        s = lax.axis_index("s")
        w = c * NS + s

        def fill_ones(i, _):
            ones_v[i] = jnp.ones((LANES,), jnp.float32)
            return 0

        lax.fori_loop(0, 128, fill_ones, 0)

        def fill_zeros(i, _):
            zeros_v[i] = jnp.zeros((LANES,), jnp.float32)
            return 0

        lax.fori_loop(0, zr, fill_zeros, 0)

        def zero_acc(k, _):
            pltpu.sync_copy(zeros_v, acc.at[pl.ds(s * rpt + k * zr, zr)])
            return 0

        lax.fori_loop(0, rpt // zr, zero_acc, 0)
        plsc.subcore_barrier()

        pltpu.sync_copy(col_hbm.at[w], col_scr)

        def body(j, _):
            pltpu.sync_copy(ones_v, acc.at[col_scr.at[j]], add=True)
            return 0

        lax.fori_loop(0, CHD, body, 0)
        plsc.subcore_barrier()
        pltpu.sync_copy(acc.at[pl.ds(s * opt, opt)],
                        out_hbm.at[c, pl.ds(s * opt, opt)])

    return deg_kernel


def _make_agg_kernel(CH, NPAD, NACC):
    rpt = NACC // NS
    zr = _largest_div(rpt, 24)
    opt = NPAD // NS
    assert CH % NB == 0
    mesh = plsc.VectorSubcoreMesh(core_axis_name="c", subcore_axis_name="s")

    @functools.partial(
        pl.kernel,
        mesh=mesh,
        out_type=jax.ShapeDtypeStruct((NC, NPAD, H), jnp.float32),
        scratch_types=[
            pltpu.VMEM((CH * CK,), jnp.int32),
            [pltpu.VMEM((1, CK), jnp.int32) for _ in range(NB)],
            [pltpu.VMEM((1, CK), jnp.int32) for _ in range(NB)],
            [pltpu.VMEM((CK, H), jnp.float32) for _ in range(NB)],
            pltpu.VMEM((zr, H), jnp.float32),
            pltpu.VMEM_SHARED((NACC, H), jnp.float32),
            pltpu.SemaphoreType.DMA,
            pltpu.SemaphoreType.DMA,
        ],
    )
    def agg_kernel(y_hbm, pk_hbm, out_hbm,
                   pk_scr, ridx, cidx, bufs, zeros_v, acc, gsem, ssem):
        c = lax.axis_index("c")
        s = lax.axis_index("s")
        w = c * NS + s

        def fill_zeros(n, _):
            i = n // (H // LANES)
            k = n % (H // LANES)
            zeros_v[i, pl.ds(k * LANES, LANES)] = jnp.zeros((LANES,),
                                                            jnp.float32)
            return 0

        lax.fori_loop(0, zr * (H // LANES), fill_zeros, 0)

        def zero_acc(k, _):
            pltpu.sync_copy(zeros_v, acc.at[pl.ds(s * rpt + k * zr, zr)])
            return 0

        lax.fori_loop(0, rpt // zr, zero_acc, 0)

        pltpu.sync_copy(pk_hbm.at[w], pk_scr)

        def unpack(j, b):
            base = pl.multiple_of(j * CK, CK)
            for k in range(CK // LANES):
                p = pk_scr[pl.ds(base + k * LANES, LANES)]
                ridx[b][0, pl.ds(k * LANES, LANES)] = p >> PSHIFT
                cidx[b][0, pl.ds(k * LANES, LANES)] = p & ((1 << PSHIFT) - 1)

        def gather(b):
            pltpu.async_copy(y_hbm.at[ridx[b].at[0]], bufs[b], gsem)

        def gather_wait(b):
            pltpu.make_async_copy(y_hbm.at[ridx[b].at[0]], bufs[b],
                                  gsem).wait()

        def scatter(b):
            pltpu.async_copy(bufs[b], acc.at[cidx[b].at[0]], ssem, add=True)

        def scatter_wait(b):
            pltpu.make_async_copy(bufs[b], acc.at[cidx[b].at[0]],
                                  ssem).wait()

        plsc.subcore_barrier()

        if LA == 0:
            def sbody(j, _):
                unpack(j, 0)
                gather(0)
                gather_wait(0)
                pltpu.sync_copy(bufs[0], acc.at[cidx[0].at[0]], add=True)
                return 0

            lax.fori_loop(0, CH, sbody, 0)
            plsc.subcore_barrier()
            pltpu.sync_copy(acc.at[pl.ds(s * opt, opt)],
                            out_hbm.at[c, pl.ds(s * opt, opt)])
            return

        for b in range(LA):
            unpack(b, b)
            gather(b)

        def body(g, _):
            j0 = g * NB
            for b in range(NB):
                j = j0 + b
                bg = (b + LA) % NB
                unpack(j + LA, bg)
                gather(bg)
                gather_wait(b)
                pltpu.sync_copy(bufs[b], acc.at[cidx[b].at[0]], add=True)
            return 0

        lax.fori_loop(0, CH // NB - 1, body, 0)
        for b in range(NB):
            j = CH - NB + b
            if j + LA < CH:
                bg = (b + LA) % NB
                unpack(j + LA, bg)
                gather(bg)
            gather_wait(b)
            pltpu.sync_copy(bufs[b], acc.at[cidx[b].at[0]], add=True)
        plsc.subcore_barrier()
        pltpu.sync_copy(acc.at[pl.ds(s * opt, opt)],
                        out_hbm.at[c, pl.ds(s * opt, opt)])

    return agg_kernel


def _prep_body(deg2, nip, xv, wh, wl, y0):
    deg = deg2[0, :, 0:1] + deg2[1, :, 0:1] + 1.0
    dinv = lax.rsqrt(deg)
    u = jnp.dot(xv[...], wh[...], precision=_HIGH,
                preferred_element_type=jnp.float32)
    y0[...] = dinv * (u + nip[...] * wl[...])


def _step_body(agg2, y, deg2, nip, wh, wl, b, w1, b1, w2p, b2p, out, ynext):
    deg = deg2[0, :, 0:1] + deg2[1, :, 0:1] + 1.0
    dinv = lax.rsqrt(deg)
    emb = dinv * (agg2[0] + agg2[1] + y[...]) + b[...]
    h = jnp.maximum(
        jnp.dot(emb, w1[...], precision=_HIGH,
                preferred_element_type=jnp.float32) + b1[...], 0.0)
    out[...] = jnp.dot(h, w2p[...], precision=_HIGH,
                       preferred_element_type=jnp.float32) + b2p[...]
    ynext[...] = dinv * (jnp.dot(emb, wh[...], precision=_HIGH,
                                 preferred_element_type=jnp.float32)
                         + nip[...] * wl[...])


def _row_spec(w):
    return pl.BlockSpec((BLK, w), lambda i: (i, 0))


def _part_spec(w):
    return pl.BlockSpec((NC, BLK, w), lambda i: (0, i, 0))


def _full_spec(h, w):
    return pl.BlockSpec((h, w), lambda i: (0, 0))


def kernel(node_input, edge_index, X_v, W, b, W1, b1, W2, b2):
    N = node_input.shape[0]
    E = edge_index.shape[1]
    NPAD = -(-N // BLK) * BLK
    NACC = NPAD + 128
    assert NPAD % NS == 0 and NACC % NS == 0
    assert NPAD < (1 << PSHIFT)
    CH = -(-(-(-E // (NW * CK))) // NB) * NB
    EPAD = NW * CH * CK
    CHD = -(-E // (NW * 128))
    EPADD = NW * CHD * 128

    row = edge_index[0]
    col = edge_index[1]
    packed = jnp.concatenate(
        [(row << PSHIFT) | col,
         jnp.full((EPAD - E,), NPAD, jnp.int32)]).reshape(NW, CH * CK)
    colp = jnp.concatenate(
        [col, jnp.full((EPADD - E,), NPAD, col.dtype)]).reshape(NW, CHD, 128)

    nip = jnp.pad(node_input, (0, NPAD - N)).reshape(NPAD, 1)
    xv = X_v.reshape(1, H)
    wh = W[:H]
    wl = W[H:H + 1]
    b_ = b.reshape(1, H)
    b1_ = b1.reshape(1, H)
    w2p = jnp.pad(W2, ((0, 0), (0, H - W2.shape[1])))
    b2p = jnp.pad(b2.reshape(1, -1), ((0, 0), (0, H - b2.shape[0])))

    deg_k = _make_deg_kernel(CHD, NPAD, NACC)
    agg_k = _make_agg_kernel(CH, NPAD, NACC)

    degp = deg_k(colp)

    grid = (NPAD // BLK,)
    y = pl.pallas_call(
        _prep_body,
        grid=grid,
        in_specs=[_part_spec(DDEG), _row_spec(1), _full_spec(1, H),
                  _full_spec(H, H), _full_spec(1, H)],
        out_specs=_row_spec(H),
        out_shape=jax.ShapeDtypeStruct((NPAD, H), jnp.float32),
    )(degp, nip, xv, wh, wl)

    step_call = pl.pallas_call(
        _step_body,
        grid=grid,
        in_specs=[_part_spec(H), _row_spec(H), _part_spec(DDEG), _row_spec(1),
                  _full_spec(H, H), _full_spec(1, H), _full_spec(1, H),
                  _full_spec(H, H), _full_spec(1, H), _full_spec(H, H),
                  _full_spec(1, H)],
        out_specs=[_row_spec(H), _row_spec(H)],
        out_shape=[jax.ShapeDtypeStruct((NPAD, H), jnp.float32),
                   jax.ShapeDtypeStruct((NPAD, H), jnp.float32)],
    )

    outs = []
    for _ in range(3):
        agg = agg_k(y, packed)
        out_full, y = step_call(agg, y, degp, nip, wh, wl, b_, W1, b1_,
                                w2p, b2p)
        outs.append(out_full[:N, 0])
    return jnp.stack(outs, axis=0)


# speedup vs baseline: 1.8461x; 1.3367x over previous
import functools

import jax
import jax.numpy as jnp
from jax import lax
from jax.experimental import pallas as pl
from jax.experimental.pallas import tpu as pltpu
from jax.experimental.pallas import tpu_sc as plsc

NC = 2
NS = 16
NW = NC * NS
LANES = 16
H = 128
BLK = 256
CK = 128
NB = 1
LA = 0
PSHIFT = 14
F0 = 0.64

_HIGH = lax.Precision.HIGHEST


def _largest_div(n, cap):
    for z in range(min(n, cap), 0, -1):
        if n % z == 0:
            return z
    return 1


DDEG = 16


def _make_deg_kernel(CHD, NPAD, NACC):
    rpt = NACC // NS
    zr = _largest_div(rpt, 96)
    opt = NPAD // NS
    mesh = plsc.VectorSubcoreMesh(core_axis_name="c", subcore_axis_name="s")

    @functools.partial(
        pl.kernel,
        mesh=mesh,
        out_type=jax.ShapeDtypeStruct((NC, NPAD, DDEG), jnp.float32),
        scratch_types=[
            pltpu.VMEM((CHD, 128), jnp.int32),
            pltpu.VMEM((128, DDEG), jnp.float32),
            pltpu.VMEM((zr, DDEG), jnp.float32),
            pltpu.VMEM_SHARED((NACC, DDEG), jnp.float32),
        ],
    )
    def deg_kernel(col_hbm, out_hbm, col_scr, ones_v, zeros_v, acc):
        c = lax.axis_index("c")
        s = lax.axis_index("s")
        w = c * NS + s

        def fill_ones(i, _):
            ones_v[i] = jnp.ones((LANES,), jnp.float32)
            return 0

        lax.fori_loop(0, 128, fill_ones, 0)

        def fill_zeros(i, _):
            zeros_v[i] = jnp.zeros((LANES,), jnp.float32)
            return 0

        lax.fori_loop(0, zr, fill_zeros, 0)

        def zero_acc(k, _):
            pltpu.sync_copy(zeros_v, acc.at[pl.ds(s * rpt + k * zr, zr)])
            return 0

        lax.fori_loop(0, rpt // zr, zero_acc, 0)
        plsc.subcore_barrier()

        pltpu.sync_copy(col_hbm.at[w], col_scr)

        def body(j, _):
            pltpu.sync_copy(ones_v, acc.at[col_scr.at[j]], add=True)
            return 0

        lax.fori_loop(0, CHD, body, 0)
        plsc.subcore_barrier()
        pltpu.sync_copy(acc.at[pl.ds(s * opt, opt)],
                        out_hbm.at[c, pl.ds(s * opt, opt)])

    return deg_kernel


def _make_agg_kernel(CH0, CH1, NPAD, NACC):
    rpt = NACC // NS
    zr = _largest_div(rpt, 24)
    opt = NPAD // NS
    mesh = plsc.VectorSubcoreMesh(core_axis_name="c", subcore_axis_name="s")

    @functools.partial(
        pl.kernel,
        mesh=mesh,
        out_type=jax.ShapeDtypeStruct((NC, NPAD, H), jnp.float32),
        scratch_types=[
            pltpu.VMEM((CH0 * CK,), jnp.int32),
            [pltpu.VMEM((1, CK), jnp.int32) for _ in range(NB)],
            [pltpu.VMEM((1, CK), jnp.int32) for _ in range(NB)],
            [pltpu.VMEM((CK, H), jnp.float32) for _ in range(NB)],
            pltpu.VMEM((zr, H), jnp.float32),
            pltpu.VMEM_SHARED((NACC, H), jnp.float32),
            pltpu.SemaphoreType.DMA,
            pltpu.SemaphoreType.DMA,
        ],
    )
    def agg_kernel(y_hbm, pk_hbm, out_hbm,
                   pk_scr, ridx, cidx, bufs, zeros_v, acc, gsem, ssem):
        c = lax.axis_index("c")
        s = lax.axis_index("s")

        def fill_zeros(n, _):
            i = n // (H // LANES)
            k = n % (H // LANES)
            zeros_v[i, pl.ds(k * LANES, LANES)] = jnp.zeros((LANES,),
                                                            jnp.float32)
            return 0

        lax.fori_loop(0, zr * (H // LANES), fill_zeros, 0)

        def zero_acc(k, _):
            pltpu.sync_copy(zeros_v, acc.at[pl.ds(s * rpt + k * zr, zr)])
            return 0

        lax.fori_loop(0, rpt // zr, zero_acc, 0)

        chn = jnp.where(c == 0, CH0, CH1)

        @pl.when(c == 0)
        def _():
            pltpu.sync_copy(pk_hbm.at[pl.ds(s * CH0 * CK, CH0 * CK)], pk_scr)

        @pl.when(c == 1)
        def _():
            pltpu.sync_copy(
                pk_hbm.at[pl.ds((NS * CH0 + s * CH1) * CK, CH1 * CK)],
                pk_scr.at[pl.ds(0, CH1 * CK)])

        def unpack(j, b):
            base = pl.multiple_of(j * CK, CK)
            for k in range(CK // LANES):
                p = pk_scr[pl.ds(base + k * LANES, LANES)]
                ridx[b][0, pl.ds(k * LANES, LANES)] = p >> PSHIFT
                cidx[b][0, pl.ds(k * LANES, LANES)] = p & ((1 << PSHIFT) - 1)

        def gather(b):
            pltpu.async_copy(y_hbm.at[ridx[b].at[0]], bufs[b], gsem)

        def gather_wait(b):
            pltpu.make_async_copy(y_hbm.at[ridx[b].at[0]], bufs[b],
                                  gsem).wait()

        def scatter(b):
            pltpu.async_copy(bufs[b], acc.at[cidx[b].at[0]], ssem, add=True)

        def scatter_wait(b):
            pltpu.make_async_copy(bufs[b], acc.at[cidx[b].at[0]],
                                  ssem).wait()

        plsc.subcore_barrier()

        def sbody(j, _):
            unpack(j, 0)
            gather(0)
            gather_wait(0)
            pltpu.sync_copy(bufs[0], acc.at[cidx[0].at[0]], add=True)
            return 0

        lax.fori_loop(0, chn, sbody, 0)
        plsc.subcore_barrier()
        pltpu.sync_copy(acc.at[pl.ds(s * opt, opt)],
                        out_hbm.at[c, pl.ds(s * opt, opt)])

    return agg_kernel


def _prep_body(deg2, nip, xv, wh, wl, y0):
    deg = deg2[0, :, 0:1] + deg2[1, :, 0:1] + 1.0
    dinv = lax.rsqrt(deg)
    u = jnp.dot(xv[...], wh[...], precision=_HIGH,
                preferred_element_type=jnp.float32)
    y0[...] = dinv * (u + nip[...] * wl[...])


def _step_body(agg2, y, deg2, nip, wh, wl, b, w1, b1, w2p, b2p, out, ynext):
    deg = deg2[0, :, 0:1] + deg2[1, :, 0:1] + 1.0
    dinv = lax.rsqrt(deg)
    emb = dinv * (agg2[0] + agg2[1] + y[...]) + b[...]
    h = jnp.maximum(
        jnp.dot(emb, w1[...], precision=_HIGH,
                preferred_element_type=jnp.float32) + b1[...], 0.0)
    out[...] = jnp.dot(h, w2p[...], precision=_HIGH,
                       preferred_element_type=jnp.float32) + b2p[...]
    ynext[...] = dinv * (jnp.dot(emb, wh[...], precision=_HIGH,
                                 preferred_element_type=jnp.float32)
                         + nip[...] * wl[...])


def _row_spec(w):
    return pl.BlockSpec((BLK, w), lambda i: (i, 0))


def _part_spec(w):
    return pl.BlockSpec((NC, BLK, w), lambda i: (0, i, 0))


def _full_spec(h, w):
    return pl.BlockSpec((h, w), lambda i: (0, 0))


def kernel(node_input, edge_index, X_v, W, b, W1, b1, W2, b2):
    N = node_input.shape[0]
    E = edge_index.shape[1]
    NPAD = -(-N // BLK) * BLK
    NACC = NPAD + 128
    assert NPAD % NS == 0 and NACC % NS == 0
    assert NPAD < (1 << PSHIFT)
    CHT = -(-E // (NS * CK))
    CH0 = int(round(CHT * F0))
    CH1 = CHT - CH0
    EPAD = NS * CHT * CK
    CHD = -(-E // (NW * 128))
    EPADD = NW * CHD * 128

    row = edge_index[0]
    col = edge_index[1]
    packed = jnp.concatenate(
        [(row << PSHIFT) | col,
         jnp.full((EPAD - E,), NPAD, jnp.int32)])
    colp = jnp.concatenate(
        [col, jnp.full((EPADD - E,), NPAD, col.dtype)]).reshape(NW, CHD, 128)

    nip = jnp.pad(node_input, (0, NPAD - N)).reshape(NPAD, 1)
    xv = X_v.reshape(1, H)
    wh = W[:H]
    wl = W[H:H + 1]
    b_ = b.reshape(1, H)
    b1_ = b1.reshape(1, H)
    w2p = jnp.pad(W2, ((0, 0), (0, H - W2.shape[1])))
    b2p = jnp.pad(b2.reshape(1, -1), ((0, 0), (0, H - b2.shape[0])))

    deg_k = _make_deg_kernel(CHD, NPAD, NACC)
    agg_k = _make_agg_kernel(CH0, CH1, NPAD, NACC)

    degp = deg_k(colp)

    grid = (NPAD // BLK,)
    y = pl.pallas_call(
        _prep_body,
        grid=grid,
        in_specs=[_part_spec(DDEG), _row_spec(1), _full_spec(1, H),
                  _full_spec(H, H), _full_spec(1, H)],
        out_specs=_row_spec(H),
        out_shape=jax.ShapeDtypeStruct((NPAD, H), jnp.float32),
    )(degp, nip, xv, wh, wl)

    step_call = pl.pallas_call(
        _step_body,
        grid=grid,
        in_specs=[_part_spec(H), _row_spec(H), _part_spec(DDEG), _row_spec(1),
                  _full_spec(H, H), _full_spec(1, H), _full_spec(1, H),
                  _full_spec(H, H), _full_spec(1, H), _full_spec(H, H),
                  _full_spec(1, H)],
        out_specs=[_row_spec(H), _row_spec(H)],
        out_shape=[jax.ShapeDtypeStruct((NPAD, H), jnp.float32),
                   jax.ShapeDtypeStruct((NPAD, H), jnp.float32)],
    )

    outs = []
    for _ in range(3):
        agg = agg_k(y, packed)
        out_full, y = step_call(agg, y, degp, nip, wh, wl, b_, W1, b1_,
                                w2p, b2p)
        outs.append(out_full[:N, 0])
    return jnp.stack(outs, axis=0)


# speedup vs baseline: 1.9239x; 1.0421x over previous
import functools

import jax
import jax.numpy as jnp
from jax import lax
from jax.experimental import pallas as pl
from jax.experimental.pallas import tpu as pltpu
from jax.experimental.pallas import tpu_sc as plsc

NC = 2
NS = 16
NW = NC * NS
LANES = 16
H = 128
BLK = 256
CK = 128
NB = 1
LA = 0
PSHIFT = 14
F0 = 0.61

_HIGH = lax.Precision.DEFAULT


def _largest_div(n, cap):
    for z in range(min(n, cap), 0, -1):
        if n % z == 0:
            return z
    return 1


DDEG = 16


def _make_deg_kernel(CHD, NPAD, NACC):
    rpt = NACC // NS
    zr = _largest_div(rpt, 96)
    opt = NPAD // NS
    mesh = plsc.VectorSubcoreMesh(core_axis_name="c", subcore_axis_name="s")

    @functools.partial(
        pl.kernel,
        mesh=mesh,
        out_type=jax.ShapeDtypeStruct((NC, NPAD, DDEG), jnp.float32),
        scratch_types=[
            pltpu.VMEM((CHD, 128), jnp.int32),
            pltpu.VMEM((128, DDEG), jnp.float32),
            pltpu.VMEM((zr, DDEG), jnp.float32),
            pltpu.VMEM_SHARED((NACC, DDEG), jnp.float32),
        ],
    )
    def deg_kernel(col_hbm, out_hbm, col_scr, ones_v, zeros_v, acc):
        c = lax.axis_index("c")
        s = lax.axis_index("s")
        w = c * NS + s

        def fill_ones(i, _):
            ones_v[i] = jnp.ones((LANES,), jnp.float32)
            return 0

        lax.fori_loop(0, 128, fill_ones, 0)

        def fill_zeros(i, _):
            zeros_v[i] = jnp.zeros((LANES,), jnp.float32)
            return 0

        lax.fori_loop(0, zr, fill_zeros, 0)

        def zero_acc(k, _):
            pltpu.sync_copy(zeros_v, acc.at[pl.ds(s * rpt + k * zr, zr)])
            return 0

        lax.fori_loop(0, rpt // zr, zero_acc, 0)
        plsc.subcore_barrier()

        pltpu.sync_copy(col_hbm.at[w], col_scr)

        def body(j, _):
            pltpu.sync_copy(ones_v, acc.at[col_scr.at[j]], add=True)
            return 0

        lax.fori_loop(0, CHD, body, 0)
        plsc.subcore_barrier()
        pltpu.sync_copy(acc.at[pl.ds(s * opt, opt)],
                        out_hbm.at[c, pl.ds(s * opt, opt)])

    return deg_kernel


def _make_agg_kernel(CH0, CH1, NPAD, NACC):
    rpt = NACC // NS
    zr = _largest_div(rpt, 24)
    opt = NPAD // NS
    mesh = plsc.VectorSubcoreMesh(core_axis_name="c", subcore_axis_name="s")

    @functools.partial(
        pl.kernel,
        mesh=mesh,
        out_type=jax.ShapeDtypeStruct((NC, NPAD, H), jnp.float32),
        scratch_types=[
            pltpu.VMEM((CH0 * CK,), jnp.int32),
            [pltpu.VMEM((1, CK), jnp.int32) for _ in range(NB)],
            [pltpu.VMEM((1, CK), jnp.int32) for _ in range(NB)],
            [pltpu.VMEM((CK, H), jnp.float32) for _ in range(NB)],
            pltpu.VMEM((zr, H), jnp.float32),
            pltpu.VMEM_SHARED((NACC, H), jnp.float32),
            pltpu.SemaphoreType.DMA,
            pltpu.SemaphoreType.DMA,
        ],
    )
    def agg_kernel(y_hbm, pk_hbm, out_hbm,
                   pk_scr, ridx, cidx, bufs, zeros_v, acc, gsem, ssem):
        c = lax.axis_index("c")
        s = lax.axis_index("s")

        def fill_zeros(n, _):
            i = n // (H // LANES)
            k = n % (H // LANES)
            zeros_v[i, pl.ds(k * LANES, LANES)] = jnp.zeros((LANES,),
                                                            jnp.float32)
            return 0

        lax.fori_loop(0, zr * (H // LANES), fill_zeros, 0)

        def zero_acc(k, _):
            pltpu.sync_copy(zeros_v, acc.at[pl.ds(s * rpt + k * zr, zr)])
            return 0

        lax.fori_loop(0, rpt // zr, zero_acc, 0)

        chn = jnp.where(c == 0, CH0, CH1)

        @pl.when(c == 0)
        def _():
            pltpu.sync_copy(pk_hbm.at[pl.ds(s * CH0 * CK, CH0 * CK)], pk_scr)

        @pl.when(c == 1)
        def _():
            pltpu.sync_copy(
                pk_hbm.at[pl.ds((NS * CH0 + s * CH1) * CK, CH1 * CK)],
                pk_scr.at[pl.ds(0, CH1 * CK)])

        def unpack(j, b):
            base = pl.multiple_of(j * CK, CK)
            for k in range(CK // LANES):
                p = pk_scr[pl.ds(base + k * LANES, LANES)]
                ridx[b][0, pl.ds(k * LANES, LANES)] = p >> PSHIFT
                cidx[b][0, pl.ds(k * LANES, LANES)] = p & ((1 << PSHIFT) - 1)

        def gather(b):
            pltpu.async_copy(y_hbm.at[ridx[b].at[0]], bufs[b], gsem)

        def gather_wait(b):
            pltpu.make_async_copy(y_hbm.at[ridx[b].at[0]], bufs[b],
                                  gsem).wait()

        def scatter(b):
            pltpu.async_copy(bufs[b], acc.at[cidx[b].at[0]], ssem, add=True)

        def scatter_wait(b):
            pltpu.make_async_copy(bufs[b], acc.at[cidx[b].at[0]],
                                  ssem).wait()

        plsc.subcore_barrier()

        def sbody(j, _):
            unpack(j, 0)
            gather(0)
            gather_wait(0)
            pltpu.sync_copy(bufs[0], acc.at[cidx[0].at[0]], add=True)
            return 0

        lax.fori_loop(0, chn, sbody, 0)
        plsc.subcore_barrier()
        pltpu.sync_copy(acc.at[pl.ds(s * opt, opt)],
                        out_hbm.at[c, pl.ds(s * opt, opt)])

    return agg_kernel


def _prep_body(deg2, nip, xv, wh, wl, y0):
    deg = deg2[0, :, 0:1] + deg2[1, :, 0:1] + 1.0
    dinv = lax.rsqrt(deg)
    u = jnp.dot(xv[...], wh[...], precision=_HIGH,
                preferred_element_type=jnp.float32)
    y0[...] = dinv * (u + nip[...] * wl[...])


def _step_body(agg2, y, deg2, nip, wh, wl, b, w1, b1, w2p, b2p, out, ynext):
    deg = deg2[0, :, 0:1] + deg2[1, :, 0:1] + 1.0
    dinv = lax.rsqrt(deg)
    emb = dinv * (agg2[0] + agg2[1] + y[...]) + b[...]
    h = jnp.maximum(
        jnp.dot(emb, w1[...], precision=_HIGH,
                preferred_element_type=jnp.float32) + b1[...], 0.0)
    out[...] = jnp.dot(h, w2p[...], precision=_HIGH,
                       preferred_element_type=jnp.float32) + b2p[...]
    ynext[...] = dinv * (jnp.dot(emb, wh[...], precision=_HIGH,
                                 preferred_element_type=jnp.float32)
                         + nip[...] * wl[...])


def _row_spec(w):
    return pl.BlockSpec((BLK, w), lambda i: (i, 0))


def _part_spec(w):
    return pl.BlockSpec((NC, BLK, w), lambda i: (0, i, 0))


def _full_spec(h, w):
    return pl.BlockSpec((h, w), lambda i: (0, 0))


def kernel(node_input, edge_index, X_v, W, b, W1, b1, W2, b2):
    N = node_input.shape[0]
    E = edge_index.shape[1]
    NPAD = -(-N // BLK) * BLK
    NACC = NPAD + 128
    assert NPAD % NS == 0 and NACC % NS == 0
    assert NPAD < (1 << PSHIFT)
    CHT = -(-E // (NS * CK))
    CH0 = int(round(CHT * F0))
    CH1 = CHT - CH0
    EPAD = NS * CHT * CK
    CHD = -(-E // (NW * 128))
    EPADD = NW * CHD * 128

    row = edge_index[0]
    col = edge_index[1]
    packed = jnp.concatenate(
        [(row << PSHIFT) | col,
         jnp.full((EPAD - E,), NPAD, jnp.int32)])
    colp = jnp.concatenate(
        [col, jnp.full((EPADD - E,), NPAD, col.dtype)]).reshape(NW, CHD, 128)

    nip = jnp.pad(node_input, (0, NPAD - N)).reshape(NPAD, 1)
    xv = X_v.reshape(1, H)
    wh = W[:H]
    wl = W[H:H + 1]
    b_ = b.reshape(1, H)
    b1_ = b1.reshape(1, H)
    w2p = jnp.pad(W2, ((0, 0), (0, H - W2.shape[1])))
    b2p = jnp.pad(b2.reshape(1, -1), ((0, 0), (0, H - b2.shape[0])))

    deg_k = _make_deg_kernel(CHD, NPAD, NACC)
    agg_k = _make_agg_kernel(CH0, CH1, NPAD, NACC)

    degp = deg_k(colp)

    grid = (NPAD // BLK,)
    y = pl.pallas_call(
        _prep_body,
        grid=grid,
        in_specs=[_part_spec(DDEG), _row_spec(1), _full_spec(1, H),
                  _full_spec(H, H), _full_spec(1, H)],
        out_specs=_row_spec(H),
        out_shape=jax.ShapeDtypeStruct((NPAD, H), jnp.float32),
    )(degp, nip, xv, wh, wl)

    step_call = pl.pallas_call(
        _step_body,
        grid=grid,
        in_specs=[_part_spec(H), _row_spec(H), _part_spec(DDEG), _row_spec(1),
                  _full_spec(H, H), _full_spec(1, H), _full_spec(1, H),
                  _full_spec(H, H), _full_spec(1, H), _full_spec(H, H),
                  _full_spec(1, H)],
        out_specs=[_row_spec(H), _row_spec(H)],
        out_shape=[jax.ShapeDtypeStruct((NPAD, H), jnp.float32),
                   jax.ShapeDtypeStruct((NPAD, H), jnp.float32)],
    )

    outs = []
    for _ in range(3):
        agg = agg_k(y, packed)
        out_full, y = step_call(agg, y, degp, nip, wh, wl, b_, W1, b1_,
                                w2p, b2p)
        outs.append(out_full[:N, 0])
    return jnp.stack(outs, axis=0)
